# Initial kernel scaffold; baseline (speedup 1.0000x reference)
import jax
import jax.numpy as jnp
from jax.experimental import pallas as pl


def kernel(sensor_batch, base_vertices, edge_index, W1, b1, g1, be1, W2, b2, g2, be2, W3, b3, g3, be3):
    raise NotImplementedError("write your pallas kernel here")


# speedup vs baseline: 74.2807x; 74.2807x over previous
import functools

import jax
import jax.numpy as jnp
from jax import lax
from jax.experimental import pallas as pl
from jax.experimental.pallas import tpu as pltpu
from jax.experimental.pallas import tpu_sc as plsc

F32 = jnp.float32
EMBED = 1024
HID = 128
N = 2048
E = 32768
B = 8
BN_EPS = 1e-5

NC = 2
NS = 16
EPT = E // NS
ROWS = 512
CHUNKS = (N // NC) // ROWS
ACC = ROWS * N
ZW = 16384


def _sc_build_adjacency(src2d, dst2d):
    mesh = plsc.VectorSubcoreMesh(core_axis_name="c", subcore_axis_name="s")

    @functools.partial(
        pl.kernel,
        out_type=[
            jax.ShapeDtypeStruct((N * N,), F32),
            jax.ShapeDtypeStruct((N,), F32),
        ],
        mesh=mesh,
        scratch_types=[
            pltpu.VMEM((EPT // 128, 128), jnp.int32),
            pltpu.VMEM((EPT // 128, 128), jnp.int32),
            pltpu.VMEM((EPT // 128 + 1, 128), jnp.int32),
            pltpu.VMEM((EPT // 128 + 1, 128), F32),
            pltpu.VMEM((ZW,), F32),
            pltpu.VMEM_SHARED((N,), F32),
            pltpu.VMEM_SHARED((ACC,), F32),
        ],
    )
    def build(src_h, dst_h, a_h, deg_h, src_v, dst_v, idx_v, val_v,
              zero_v, deg_s, acc_s):
        cid = lax.axis_index("c")
        sid = lax.axis_index("s")
        nrows = EPT // 128

        pltpu.sync_copy(src_h.at[pl.ds(sid * nrows, nrows), :], src_v)
        pltpu.sync_copy(dst_h.at[pl.ds(sid * nrows, nrows), :], dst_v)

        def zfill(i, _):
            zero_v[pl.ds(i * 16, 16)] = jnp.zeros((16,), F32)
            return 0
        lax.fori_loop(0, ZW // 16, zfill, 0, unroll=8)

        for j in range(nrows):
            def ofill(c, _, j=j):
                val_v[j, pl.ds(c * 16, 16)] = jnp.full((16,), 1.0, F32)
                return 0
            lax.fori_loop(0, 8, ofill, 0, unroll=8)

        @pl.when(sid == 0)
        def _():
            pltpu.sync_copy(zero_v.at[pl.ds(0, N)], deg_s)
        plsc.subcore_barrier()

        for j in range(nrows):
            pltpu.sync_copy(val_v.at[j], deg_s.at[dst_v.at[j]], add=True)
        plsc.subcore_barrier()

        @pl.when((cid == 0) & (sid == 0))
        def _():
            pltpu.sync_copy(deg_s, deg_h)

        for cc in range(CHUNKS):
            lo = cid * (N // NC) + cc * ROWS

            for r in range(ACC // NS // ZW):
                pltpu.sync_copy(
                    zero_v, acc_s.at[pl.ds(sid * (ACC // NS) + r * ZW, ZW)])
            plsc.subcore_barrier()

            for j in range(nrows):
                def grp(c, _, j=j):
                    sl = pl.ds(c * 16, 16)
                    s = src_v[j, sl]
                    d = dst_v[j, sl]
                    m = (d >= lo) & (d < lo + ROWS)
                    idx_v[j, sl] = jnp.where(m, (d - lo) * N + s, 0)
                    val_v[j, sl] = jnp.where(m, jnp.float32(1.0),
                                             jnp.float32(0.0))
                    return 0
                lax.fori_loop(0, 8, grp, 0)
                pltpu.sync_copy(val_v.at[j], acc_s.at[idx_v.at[j]], add=True)

            for q in range(8):
                sl = pl.ds(q * 16, 16)
                if q < 2:
                    i16 = lo + sid * 32 + q * 16 + lax.iota(jnp.int32, 16)
                    idx_v[nrows, sl] = (i16 - lo) * N + i16
                    val_v[nrows, sl] = jnp.full((16,), 1.0, F32)
                else:
                    idx_v[nrows, sl] = jnp.zeros((16,), jnp.int32)
                    val_v[nrows, sl] = jnp.zeros((16,), F32)
            pltpu.sync_copy(val_v.at[nrows], acc_s.at[idx_v.at[nrows]],
                            add=True)
            plsc.subcore_barrier()

            span = ACC // NS
            pltpu.sync_copy(
                acc_s.at[pl.ds(sid * span, span)],
                a_h.at[pl.ds(lo * N + sid * span, span)])
            plsc.subcore_barrier()

    return build(src2d, dst2d)


_TCPARAMS = pltpu.CompilerParams(dimension_semantics=("arbitrary",))
RB = 256
G = N // RB


def _dinv(deg):
    return lax.rsqrt(deg + jnp.float32(1.0))


def _k1_body(base_ref, w_ref, sens_ref, degb_ref, degf_ref, h_ref, s_ref):
    dv = _dinv(degb_ref[...])
    h = jnp.dot(base_ref[...], w_ref[...], preferred_element_type=F32)
    h_ref[...] = h * dv[:, None]
    @pl.when(pl.program_id(0) == 0)
    def _():
        dvf = _dinv(degf_ref[...])
        sel = lax.broadcasted_iota(jnp.int32, (1, N), 1) == (N - 2)
        d2046 = jnp.sum(jnp.where(sel, dvf[None, :], jnp.float32(0.0)))
        s_ref[...] = jnp.dot(sens_ref[...], w_ref[...],
                             preferred_element_type=F32) * d2046


def _layer1_h(base_p, W1, sensor, deg):
    return pl.pallas_call(
        _k1_body,
        grid=(G,),
        in_specs=[
            pl.BlockSpec((RB, EMBED), lambda i: (i, 0)),
            pl.BlockSpec((EMBED, HID), lambda i: (0, 0)),
            pl.BlockSpec((B, EMBED), lambda i: (0, 0)),
            pl.BlockSpec((RB,), lambda i: (i,)),
            pl.BlockSpec((N,), lambda i: (0,)),
        ],
        out_specs=[
            pl.BlockSpec((RB, HID), lambda i: (i, 0)),
            pl.BlockSpec((B, HID), lambda i: (0, 0)),
        ],
        out_shape=[
            jax.ShapeDtypeStruct((N, HID), F32),
            jax.ShapeDtypeStruct((B, HID), F32),
        ],
        compiler_params=_TCPARAMS,
    )(base_p, W1, sensor, deg, deg)


def _k2l1_body(a_ref, h_ref, s_ref, degb_ref, y_ref, sum_ref, sq_ref):
    dv = _dinv(degb_ref[...])
    z = jnp.dot(a_ref[...], h_ref[...], preferred_element_type=F32)
    u = a_ref[:, N - 2:N - 1] * dv[:, None]
    z = z * dv[:, None]
    s = s_ref[...]
    for b in range(B):
        y_ref[:, b * HID:(b + 1) * HID] = z + u * s[b:b + 1, :]
    y = y_ref[...]
    sum_ref[0, ...] = jnp.sum(y, axis=0, keepdims=True)
    sq_ref[0, ...] = jnp.sum(y * y, axis=0, keepdims=True)


def _k2_body(a_ref, h_ref, degb_ref, y_ref, sum_ref, sq_ref):
    dv = _dinv(degb_ref[...])
    y = jnp.dot(a_ref[...], h_ref[...], preferred_element_type=F32)
    y = y * dv[:, None]
    y_ref[...] = y
    sum_ref[0, ...] = jnp.sum(y, axis=0, keepdims=True)
    sq_ref[0, ...] = jnp.sum(y * y, axis=0, keepdims=True)


def _agg_l1(A, H1, S, deg):
    return pl.pallas_call(
        _k2l1_body,
        grid=(G,),
        in_specs=[
            pl.BlockSpec((RB, N), lambda i: (i, 0)),
            pl.BlockSpec((N, HID), lambda i: (0, 0)),
            pl.BlockSpec((B, HID), lambda i: (0, 0)),
            pl.BlockSpec((RB,), lambda i: (i,)),
        ],
        out_specs=[
            pl.BlockSpec((RB, B * HID), lambda i: (i, 0)),
            pl.BlockSpec((1, 1, B * HID), lambda i: (i, 0, 0)),
            pl.BlockSpec((1, 1, B * HID), lambda i: (i, 0, 0)),
        ],
        out_shape=[
            jax.ShapeDtypeStruct((N, B * HID), F32),
            jax.ShapeDtypeStruct((G, 1, B * HID), F32),
            jax.ShapeDtypeStruct((G, 1, B * HID), F32),
        ],
        compiler_params=_TCPARAMS,
    )(A, H1, S, deg)


def _agg(A, H, deg):
    return pl.pallas_call(
        _k2_body,
        grid=(G,),
        in_specs=[
            pl.BlockSpec((RB, N), lambda i: (i, 0)),
            pl.BlockSpec((N, B * HID), lambda i: (0, 0)),
            pl.BlockSpec((RB,), lambda i: (i,)),
        ],
        out_specs=[
            pl.BlockSpec((RB, B * HID), lambda i: (i, 0)),
            pl.BlockSpec((1, 1, B * HID), lambda i: (i, 0, 0)),
            pl.BlockSpec((1, 1, B * HID), lambda i: (i, 0, 0)),
        ],
        out_shape=[
            jax.ShapeDtypeStruct((N, B * HID), F32),
            jax.ShapeDtypeStruct((G, 1, B * HID), F32),
            jax.ShapeDtypeStruct((G, 1, B * HID), F32),
        ],
        compiler_params=_TCPARAMS,
    )(A, H, deg)


def _bn_scale_shift(sum_ref, sq_ref, g_ref, be_ref):
    cs = jnp.sum(sum_ref[...], axis=(0, 1)).reshape(B, HID)
    cq = jnp.sum(sq_ref[...], axis=(0, 1)).reshape(B, HID)
    inv_n = jnp.float32(1.0 / (B * N))
    mu = jnp.sum(cs, axis=0) * inv_n
    ex2 = jnp.sum(cq, axis=0) * inv_n
    var = ex2 - mu * mu
    scale = lax.rsqrt(var + BN_EPS) * g_ref[...]
    shift = be_ref[...] - mu * scale
    return scale, shift


def _k3_body(y_ref, sum_ref, sq_ref, g_ref, be_ref, w_ref, degb_ref, h_ref):
    scale, shift = _bn_scale_shift(sum_ref, sq_ref, g_ref, be_ref)
    dv = _dinv(degb_ref[...])
    y = y_ref[...]
    w = w_ref[...]
    for b in range(B):
        yb = y[:, b * HID:(b + 1) * HID] * scale[None, :] + shift[None, :]
        xb = jnp.where(yb > 0, yb, jnp.exp(yb) - jnp.float32(1.0))
        h_ref[:, b * HID:(b + 1) * HID] = jnp.dot(
            xb, w, preferred_element_type=F32) * dv[:, None]


def _k3f_body(y_ref, sum_ref, sq_ref, g_ref, be_ref, x_ref):
    scale, shift = _bn_scale_shift(sum_ref, sq_ref, g_ref, be_ref)
    y = y_ref[...]
    for b in range(B):
        yb = y[:, b * HID:(b + 1) * HID] * scale[None, :] + shift[None, :]
        x_ref[:, b * HID:(b + 1) * HID] = jnp.where(
            yb > 0, yb, jnp.exp(yb) - jnp.float32(1.0))


def _bn_elu_mm(Y, ssum, ssq, g, be, Wn, deg):
    return pl.pallas_call(
        _k3_body,
        grid=(G,),
        in_specs=[
            pl.BlockSpec((RB, B * HID), lambda i: (i, 0)),
            pl.BlockSpec((G, 1, B * HID), lambda i: (0, 0, 0)),
            pl.BlockSpec((G, 1, B * HID), lambda i: (0, 0, 0)),
            pl.BlockSpec((HID,), lambda i: (0,)),
            pl.BlockSpec((HID,), lambda i: (0,)),
            pl.BlockSpec((HID, HID), lambda i: (0, 0)),
            pl.BlockSpec((RB,), lambda i: (i,)),
        ],
        out_specs=pl.BlockSpec((RB, B * HID), lambda i: (i, 0)),
        out_shape=jax.ShapeDtypeStruct((N, B * HID), F32),
        compiler_params=_TCPARAMS,
    )(Y, ssum, ssq, g, be, Wn, deg)


def _bn_elu(Y, ssum, ssq, g, be):
    return pl.pallas_call(
        _k3f_body,
        grid=(G,),
        in_specs=[
            pl.BlockSpec((RB, B * HID), lambda i: (i, 0)),
            pl.BlockSpec((G, 1, B * HID), lambda i: (0, 0, 0)),
            pl.BlockSpec((G, 1, B * HID), lambda i: (0, 0, 0)),
            pl.BlockSpec((HID,), lambda i: (0,)),
            pl.BlockSpec((HID,), lambda i: (0,)),
        ],
        out_specs=pl.BlockSpec((RB, B * HID), lambda i: (i, 0)),
        out_shape=jax.ShapeDtypeStruct((N, B * HID), F32),
        compiler_params=_TCPARAMS,
    )(Y, ssum, ssq, g, be)


def kernel(sensor_batch, base_vertices, edge_index,
           W1, b1, g1, be1, W2, b2, g2, be2, W3, b3, g3, be3):
    src2d = edge_index[0].reshape(E // 128, 128)
    dst2d = edge_index[1].reshape(E // 128, 128)
    A_flat, deg = _sc_build_adjacency(src2d, dst2d)
    A = A_flat.reshape(N, N)

    base_p = jnp.concatenate(
        [base_vertices, jnp.zeros((2, EMBED), F32)], axis=0)
    H1, S = _layer1_h(base_p, W1, sensor_batch, deg)

    Y1, s1, q1 = _agg_l1(A, H1, S, deg)
    H2 = _bn_elu_mm(Y1, s1, q1, g1, be1, W2, deg)
    Y2, s2, q2 = _agg(A, H2, deg)
    H3 = _bn_elu_mm(Y2, s2, q2, g2, be2, W3, deg)
    Y3, s3, q3 = _agg(A, H3, deg)
    X3 = _bn_elu(Y3, s3, q3, g3, be3)

    return X3.reshape(N, B, HID).transpose(1, 0, 2)


# speedup vs baseline: 87.7383x; 1.1812x over previous
import functools

import jax
import jax.numpy as jnp
from jax import lax
from jax.experimental import pallas as pl
from jax.experimental.pallas import tpu as pltpu
from jax.experimental.pallas import tpu_sc as plsc

F32 = jnp.float32
EMBED = 1024
HID = 128
N = 2048
E = 32768
B = 8
BN_EPS = 1e-5

NC = 2
NS = 16
EPT = E // NS
ROWS = 512
CHUNKS = (N // NC) // ROWS
ACC = ROWS * N
ZW = 16384


def _sc_build_adjacency(src2d, dst2d):
    mesh = plsc.VectorSubcoreMesh(core_axis_name="c", subcore_axis_name="s")

    @functools.partial(
        pl.kernel,
        out_type=[
            jax.ShapeDtypeStruct((N * N,), F32),
            jax.ShapeDtypeStruct((N,), F32),
        ],
        mesh=mesh,
        scratch_types=[
            pltpu.VMEM((EPT // 128, 128), jnp.int32),
            pltpu.VMEM((EPT // 128, 128), jnp.int32),
            pltpu.VMEM((EPT // 128 + 1, 128), jnp.int32),
            pltpu.VMEM((EPT // 128 + 1, 128), F32),
            pltpu.VMEM((ZW,), F32),
            pltpu.VMEM_SHARED((N,), F32),
            pltpu.VMEM_SHARED((ACC,), F32),
        ],
    )
    def build(src_h, dst_h, a_h, deg_h, src_v, dst_v, idx_v, val_v,
              zero_v, deg_s, acc_s):
        cid = lax.axis_index("c")
        sid = lax.axis_index("s")
        nrows = EPT // 128

        pltpu.sync_copy(src_h.at[pl.ds(sid * nrows, nrows), :], src_v)
        pltpu.sync_copy(dst_h.at[pl.ds(sid * nrows, nrows), :], dst_v)

        def zfill(i, _):
            zero_v[pl.ds(i * 16, 16)] = jnp.zeros((16,), F32)
            return 0
        lax.fori_loop(0, ZW // 16, zfill, 0, unroll=8)

        for j in range(nrows):
            def ofill(c, _, j=j):
                val_v[j, pl.ds(c * 16, 16)] = jnp.full((16,), 1.0, F32)
                return 0
            lax.fori_loop(0, 8, ofill, 0, unroll=8)

        @pl.when(sid == 0)
        def _():
            pltpu.sync_copy(zero_v.at[pl.ds(0, N)], deg_s)
        plsc.subcore_barrier()

        for j in range(nrows):
            pltpu.sync_copy(val_v.at[j], deg_s.at[dst_v.at[j]], add=True)
        plsc.subcore_barrier()

        @pl.when((cid == 0) & (sid == 0))
        def _():
            pltpu.sync_copy(deg_s, deg_h)

        for cc in range(CHUNKS):
            lo = cid * (N // NC) + cc * ROWS

            for r in range(ACC // NS // ZW):
                pltpu.sync_copy(
                    zero_v, acc_s.at[pl.ds(sid * (ACC // NS) + r * ZW, ZW)])
            plsc.subcore_barrier()

            for j in range(nrows):
                def grp(c, _, j=j):
                    sl = pl.ds(c * 16, 16)
                    s = src_v[j, sl]
                    d = dst_v[j, sl]
                    m = (d >= lo) & (d < lo + ROWS)
                    idx_v[j, sl] = jnp.where(m, (d - lo) * N + s, 0)
                    val_v[j, sl] = jnp.where(m, jnp.float32(1.0),
                                             jnp.float32(0.0))
                    return 0
                lax.fori_loop(0, 8, grp, 0)
                pltpu.sync_copy(val_v.at[j], acc_s.at[idx_v.at[j]], add=True)

            for q in range(8):
                sl = pl.ds(q * 16, 16)
                if q < 2:
                    i16 = lo + sid * 32 + q * 16 + lax.iota(jnp.int32, 16)
                    idx_v[nrows, sl] = (i16 - lo) * N + i16
                    val_v[nrows, sl] = jnp.full((16,), 1.0, F32)
                else:
                    idx_v[nrows, sl] = jnp.zeros((16,), jnp.int32)
                    val_v[nrows, sl] = jnp.zeros((16,), F32)
            pltpu.sync_copy(val_v.at[nrows], acc_s.at[idx_v.at[nrows]],
                            add=True)
            plsc.subcore_barrier()

            span = ACC // NS
            pltpu.sync_copy(
                acc_s.at[pl.ds(sid * span, span)],
                a_h.at[pl.ds(lo * N + sid * span, span)])
            plsc.subcore_barrier()

    return build(src2d, dst2d)


_TCPARAMS = pltpu.CompilerParams(dimension_semantics=("arbitrary",))
RB = 256
G = N // RB


def _dinv(deg):
    return lax.rsqrt(deg + jnp.float32(1.0))


def _mega_body(a_ref, base_ref, w1_ref, sens_ref, deg_ref,
               w2_ref, w3_ref, g1_ref, be1_ref, g2_ref, be2_ref,
               g3_ref, be3_ref, out_ref,
               h_ref, y_ref, s_ref, sum_ref, sq_ref):
    step = pl.program_id(0)
    phase = step // G
    i = step % G

    def dv_blk():
        return _dinv(deg_ref[pl.ds(i * RB, RB)])

    def accum_stats(y):
        prev_s = jnp.where(i == 0, jnp.float32(0.0), sum_ref[...])
        prev_q = jnp.where(i == 0, jnp.float32(0.0), sq_ref[...])
        sum_ref[...] = prev_s + jnp.sum(y, axis=0, keepdims=True)
        sq_ref[...] = prev_q + jnp.sum(y * y, axis=0, keepdims=True)

    @pl.when(phase == 0)
    def _p0():
        dv = dv_blk()
        h = jnp.dot(base_ref[...], w1_ref[...], preferred_element_type=F32)
        h_ref[pl.ds(i * RB, RB), :HID] = h * dv[:, None]
        @pl.when(i == 0)
        def _():
            dvf = _dinv(deg_ref[...])
            sel = lax.broadcasted_iota(jnp.int32, (1, N), 1) == (N - 2)
            d2046 = jnp.sum(jnp.where(sel, dvf[None, :], jnp.float32(0.0)))
            s_ref[...] = jnp.dot(sens_ref[...], w1_ref[...],
                                 preferred_element_type=F32) * d2046

    @pl.when(phase == 1)
    def _p1():
        dv = dv_blk()
        a_blk = a_ref[pl.ds(i * RB, RB), :]
        z = jnp.dot(a_blk, h_ref[:, :HID],
                    preferred_element_type=F32) * dv[:, None]
        u = a_blk[:, N - 2:N - 1] * dv[:, None]
        s = s_ref[...]
        for b in range(B):
            y_ref[pl.ds(i * RB, RB), b * HID:(b + 1) * HID] = (
                z + u * s[b:b + 1, :])
        accum_stats(y_ref[pl.ds(i * RB, RB), :])

    def bn_elu_mm(w_ref_n, g_r, be_r):
        scale, shift = _bn_scale_shift2(sum_ref, sq_ref, g_r, be_r)
        dv = dv_blk()
        y = y_ref[pl.ds(i * RB, RB), :]
        w = w_ref_n[...]
        for b in range(B):
            yb = (y[:, b * HID:(b + 1) * HID] * scale[None, :]
                  + shift[None, :])
            xb = jnp.where(yb > 0, yb, jnp.exp(yb) - jnp.float32(1.0))
            h_ref[pl.ds(i * RB, RB), b * HID:(b + 1) * HID] = jnp.dot(
                xb, w, preferred_element_type=F32) * dv[:, None]

    def agg():
        dv = dv_blk()
        y = jnp.dot(a_ref[pl.ds(i * RB, RB), :], h_ref[...],
                    preferred_element_type=F32) * dv[:, None]
        y_ref[pl.ds(i * RB, RB), :] = y
        accum_stats(y)

    @pl.when(phase == 2)
    def _p2():
        bn_elu_mm(w2_ref, g1_ref, be1_ref)

    @pl.when(phase == 3)
    def _p3():
        agg()

    @pl.when(phase == 4)
    def _p4():
        bn_elu_mm(w3_ref, g2_ref, be2_ref)

    @pl.when(phase == 5)
    def _p5():
        agg()

    @pl.when(phase == 6)
    def _p6():
        scale, shift = _bn_scale_shift2(sum_ref, sq_ref, g3_ref, be3_ref)
        y = y_ref[pl.ds(i * RB, RB), :]
        for b in range(B):
            yb = (y[:, b * HID:(b + 1) * HID] * scale[None, :]
                  + shift[None, :])
            out_ref[:, b * HID:(b + 1) * HID] = jnp.where(
                yb > 0, yb, jnp.exp(yb) - jnp.float32(1.0))


def _bn_scale_shift2(sum_ref, sq_ref, g_ref, be_ref):
    cs = sum_ref[...].reshape(B, HID)
    cq = sq_ref[...].reshape(B, HID)
    inv_n = jnp.float32(1.0 / (B * N))
    mu = jnp.sum(cs, axis=0) * inv_n
    ex2 = jnp.sum(cq, axis=0) * inv_n
    var = ex2 - mu * mu
    scale = lax.rsqrt(var + BN_EPS) * g_ref[...]
    shift = be_ref[...] - mu * scale
    return scale, shift


def _mega(A, base_p, W1, sensor, deg, W2, W3, g1, be1, g2, be2, g3, be3):
    c0 = lambda s: (0, 0)
    c1 = lambda s: (0,)
    return pl.pallas_call(
        _mega_body,
        grid=(7 * G,),
        in_specs=[
            pl.BlockSpec((N, N), c0),
            pl.BlockSpec((RB, EMBED), lambda s: (jnp.minimum(s, G - 1), 0)),
            pl.BlockSpec((EMBED, HID), c0),
            pl.BlockSpec((B, EMBED), c0),
            pl.BlockSpec((N,), c1),
            pl.BlockSpec((HID, HID), c0),
            pl.BlockSpec((HID, HID), c0),
            pl.BlockSpec((HID,), c1),
            pl.BlockSpec((HID,), c1),
            pl.BlockSpec((HID,), c1),
            pl.BlockSpec((HID,), c1),
            pl.BlockSpec((HID,), c1),
            pl.BlockSpec((HID,), c1),
        ],
        out_specs=pl.BlockSpec(
            (RB, B * HID), lambda s: (jnp.clip(s - 6 * G, 0, G - 1), 0)),
        out_shape=jax.ShapeDtypeStruct((N, B * HID), F32),
        scratch_shapes=[
            pltpu.VMEM((N, B * HID), F32),
            pltpu.VMEM((N, B * HID), F32),
            pltpu.VMEM((B, HID), F32),
            pltpu.VMEM((1, B * HID), F32),
            pltpu.VMEM((1, B * HID), F32),
        ],
        compiler_params=_TCPARAMS,
    )(A, base_p, W1, sensor, deg, W2, W3, g1, be1, g2, be2, g3, be3)


def _k1_body(base_ref, w_ref, sens_ref, degb_ref, degf_ref, h_ref, s_ref):
    dv = _dinv(degb_ref[...])
    h = jnp.dot(base_ref[...], w_ref[...], preferred_element_type=F32)
    h_ref[...] = h * dv[:, None]
    @pl.when(pl.program_id(0) == 0)
    def _():
        dvf = _dinv(degf_ref[...])
        sel = lax.broadcasted_iota(jnp.int32, (1, N), 1) == (N - 2)
        d2046 = jnp.sum(jnp.where(sel, dvf[None, :], jnp.float32(0.0)))
        s_ref[...] = jnp.dot(sens_ref[...], w_ref[...],
                             preferred_element_type=F32) * d2046


def _layer1_h(base_p, W1, sensor, deg):
    return pl.pallas_call(
        _k1_body,
        grid=(G,),
        in_specs=[
            pl.BlockSpec((RB, EMBED), lambda i: (i, 0)),
            pl.BlockSpec((EMBED, HID), lambda i: (0, 0)),
            pl.BlockSpec((B, EMBED), lambda i: (0, 0)),
            pl.BlockSpec((RB,), lambda i: (i,)),
            pl.BlockSpec((N,), lambda i: (0,)),
        ],
        out_specs=[
            pl.BlockSpec((RB, HID), lambda i: (i, 0)),
            pl.BlockSpec((B, HID), lambda i: (0, 0)),
        ],
        out_shape=[
            jax.ShapeDtypeStruct((N, HID), F32),
            jax.ShapeDtypeStruct((B, HID), F32),
        ],
        compiler_params=_TCPARAMS,
    )(base_p, W1, sensor, deg, deg)


def _k2l1_body(a_ref, h_ref, s_ref, degb_ref, y_ref, sum_ref, sq_ref):
    dv = _dinv(degb_ref[...])
    z = jnp.dot(a_ref[...], h_ref[...], preferred_element_type=F32)
    u = a_ref[:, N - 2:N - 1] * dv[:, None]
    z = z * dv[:, None]
    s = s_ref[...]
    for b in range(B):
        y_ref[:, b * HID:(b + 1) * HID] = z + u * s[b:b + 1, :]
    y = y_ref[...]
    sum_ref[0, ...] = jnp.sum(y, axis=0, keepdims=True)
    sq_ref[0, ...] = jnp.sum(y * y, axis=0, keepdims=True)


def _k2_body(a_ref, h_ref, degb_ref, y_ref, sum_ref, sq_ref):
    dv = _dinv(degb_ref[...])
    y = jnp.dot(a_ref[...], h_ref[...], preferred_element_type=F32)
    y = y * dv[:, None]
    y_ref[...] = y
    sum_ref[0, ...] = jnp.sum(y, axis=0, keepdims=True)
    sq_ref[0, ...] = jnp.sum(y * y, axis=0, keepdims=True)


def _agg_l1(A, H1, S, deg):
    return pl.pallas_call(
        _k2l1_body,
        grid=(G,),
        in_specs=[
            pl.BlockSpec((RB, N), lambda i: (i, 0)),
            pl.BlockSpec((N, HID), lambda i: (0, 0)),
            pl.BlockSpec((B, HID), lambda i: (0, 0)),
            pl.BlockSpec((RB,), lambda i: (i,)),
        ],
        out_specs=[
            pl.BlockSpec((RB, B * HID), lambda i: (i, 0)),
            pl.BlockSpec((1, 1, B * HID), lambda i: (i, 0, 0)),
            pl.BlockSpec((1, 1, B * HID), lambda i: (i, 0, 0)),
        ],
        out_shape=[
            jax.ShapeDtypeStruct((N, B * HID), F32),
            jax.ShapeDtypeStruct((G, 1, B * HID), F32),
            jax.ShapeDtypeStruct((G, 1, B * HID), F32),
        ],
        compiler_params=_TCPARAMS,
    )(A, H1, S, deg)


def _agg(A, H, deg):
    return pl.pallas_call(
        _k2_body,
        grid=(G,),
        in_specs=[
            pl.BlockSpec((RB, N), lambda i: (i, 0)),
            pl.BlockSpec((N, B * HID), lambda i: (0, 0)),
            pl.BlockSpec((RB,), lambda i: (i,)),
        ],
        out_specs=[
            pl.BlockSpec((RB, B * HID), lambda i: (i, 0)),
            pl.BlockSpec((1, 1, B * HID), lambda i: (i, 0, 0)),
            pl.BlockSpec((1, 1, B * HID), lambda i: (i, 0, 0)),
        ],
        out_shape=[
            jax.ShapeDtypeStruct((N, B * HID), F32),
            jax.ShapeDtypeStruct((G, 1, B * HID), F32),
            jax.ShapeDtypeStruct((G, 1, B * HID), F32),
        ],
        compiler_params=_TCPARAMS,
    )(A, H, deg)


def _bn_scale_shift(sum_ref, sq_ref, g_ref, be_ref):
    cs = jnp.sum(sum_ref[...], axis=(0, 1)).reshape(B, HID)
    cq = jnp.sum(sq_ref[...], axis=(0, 1)).reshape(B, HID)
    inv_n = jnp.float32(1.0 / (B * N))
    mu = jnp.sum(cs, axis=0) * inv_n
    ex2 = jnp.sum(cq, axis=0) * inv_n
    var = ex2 - mu * mu
    scale = lax.rsqrt(var + BN_EPS) * g_ref[...]
    shift = be_ref[...] - mu * scale
    return scale, shift


def _k3_body(y_ref, sum_ref, sq_ref, g_ref, be_ref, w_ref, degb_ref, h_ref):
    scale, shift = _bn_scale_shift(sum_ref, sq_ref, g_ref, be_ref)
    dv = _dinv(degb_ref[...])
    y = y_ref[...]
    w = w_ref[...]
    for b in range(B):
        yb = y[:, b * HID:(b + 1) * HID] * scale[None, :] + shift[None, :]
        xb = jnp.where(yb > 0, yb, jnp.exp(yb) - jnp.float32(1.0))
        h_ref[:, b * HID:(b + 1) * HID] = jnp.dot(
            xb, w, preferred_element_type=F32) * dv[:, None]


def _k3f_body(y_ref, sum_ref, sq_ref, g_ref, be_ref, x_ref):
    scale, shift = _bn_scale_shift(sum_ref, sq_ref, g_ref, be_ref)
    y = y_ref[...]
    for b in range(B):
        yb = y[:, b * HID:(b + 1) * HID] * scale[None, :] + shift[None, :]
        x_ref[:, b * HID:(b + 1) * HID] = jnp.where(
            yb > 0, yb, jnp.exp(yb) - jnp.float32(1.0))


def _bn_elu_mm(Y, ssum, ssq, g, be, Wn, deg):
    return pl.pallas_call(
        _k3_body,
        grid=(G,),
        in_specs=[
            pl.BlockSpec((RB, B * HID), lambda i: (i, 0)),
            pl.BlockSpec((G, 1, B * HID), lambda i: (0, 0, 0)),
            pl.BlockSpec((G, 1, B * HID), lambda i: (0, 0, 0)),
            pl.BlockSpec((HID,), lambda i: (0,)),
            pl.BlockSpec((HID,), lambda i: (0,)),
            pl.BlockSpec((HID, HID), lambda i: (0, 0)),
            pl.BlockSpec((RB,), lambda i: (i,)),
        ],
        out_specs=pl.BlockSpec((RB, B * HID), lambda i: (i, 0)),
        out_shape=jax.ShapeDtypeStruct((N, B * HID), F32),
        compiler_params=_TCPARAMS,
    )(Y, ssum, ssq, g, be, Wn, deg)


def _bn_elu(Y, ssum, ssq, g, be):
    return pl.pallas_call(
        _k3f_body,
        grid=(G,),
        in_specs=[
            pl.BlockSpec((RB, B * HID), lambda i: (i, 0)),
            pl.BlockSpec((G, 1, B * HID), lambda i: (0, 0, 0)),
            pl.BlockSpec((G, 1, B * HID), lambda i: (0, 0, 0)),
            pl.BlockSpec((HID,), lambda i: (0,)),
            pl.BlockSpec((HID,), lambda i: (0,)),
        ],
        out_specs=pl.BlockSpec((RB, B * HID), lambda i: (i, 0)),
        out_shape=jax.ShapeDtypeStruct((N, B * HID), F32),
        compiler_params=_TCPARAMS,
    )(Y, ssum, ssq, g, be)


def kernel(sensor_batch, base_vertices, edge_index,
           W1, b1, g1, be1, W2, b2, g2, be2, W3, b3, g3, be3):
    src2d = edge_index[0].reshape(E // 128, 128)
    dst2d = edge_index[1].reshape(E // 128, 128)
    A_flat, deg = _sc_build_adjacency(src2d, dst2d)
    A = A_flat.reshape(N, N)

    base_p = jnp.concatenate(
        [base_vertices, jnp.zeros((2, EMBED), F32)], axis=0)
    X3 = _mega(A, base_p, W1, sensor_batch, deg,
               W2, W3, g1, be1, g2, be2, g3, be3)

    return X3.reshape(N, B, HID).transpose(1, 0, 2)


# speedup vs baseline: 94.9874x; 1.0826x over previous
import functools

import jax
import jax.numpy as jnp
from jax import lax
from jax.experimental import pallas as pl
from jax.experimental.pallas import tpu as pltpu
from jax.experimental.pallas import tpu_sc as plsc

F32 = jnp.float32
EMBED = 1024
HID = 128
N = 2048
E = 32768
B = 8
BN_EPS = 1e-5

NC = 2
NS = 16


PASSES = ((0, 48), (1536, 16))
ECH = 8192


def _sc_build_adjacency(src1d, dst1d):
    mesh = plsc.VectorSubcoreMesh(core_axis_name="c", subcore_axis_name="s")

    @functools.partial(
        pl.kernel,
        out_type=jax.ShapeDtypeStruct((N * N,), F32),
        mesh=mesh,
        scratch_types=[
            pltpu.VMEM((ECH,), jnp.int32),
            pltpu.VMEM((ECH,), jnp.int32),
            pltpu.VMEM((PASSES[0][1] * N,), F32),
        ],
        compiler_params=pltpu.CompilerParams(needs_layout_passes=False),
    )
    def build(src_h, dst_h, a_h, src_v, dst_v, acc_v):
        cid = lax.axis_index("c")
        sid = lax.axis_index("s")
        wid = cid * NS + sid

        for base_row, rpt in PASSES:
            lo = base_row + wid * rpt
            nw = rpt * N

            def zfill(i, _):
                acc_v[pl.ds(i * 16, 16)] = jnp.zeros((16,), F32)
                return 0
            lax.fori_loop(0, nw // 16, zfill, 0, unroll=8)

            ones = jnp.full((16,), 1.0, F32)
            tmask = jnp.full((16,), True)
            for q in range(rpt // 16):
                rel = q * 16 + lax.iota(jnp.int32, 16)
                plsc.addupdate_scatter(
                    acc_v, [rel * (N + 1) + lo], ones, mask=tmask)

            for ch in range(E // ECH):
                pltpu.sync_copy(src_h.at[pl.ds(ch * ECH, ECH)], src_v)
                pltpu.sync_copy(dst_h.at[pl.ds(ch * ECH, ECH)], dst_v)

                def grp(g, _):
                    sl = pl.ds(g * 16, 16)
                    s = src_v[sl]
                    rel = dst_v[sl] - lo
                    m = (rel >= 0) & (rel < rpt)
                    idx = jnp.where(m, rel * N + s, 0)
                    plsc.addupdate_scatter(
                        acc_v, [idx], jnp.full((16,), 1.0, F32), mask=m)
                    return 0
                lax.fori_loop(0, ECH // 16, grp, 0)

            pltpu.sync_copy(acc_v.at[pl.ds(0, nw)],
                            a_h.at[pl.ds(lo * N, nw)])

    return build(src1d, dst1d)


_TCPARAMS = pltpu.CompilerParams(dimension_semantics=("arbitrary",))
RB = 256
G = N // RB


def _dinv(deg):
    return lax.rsqrt(deg + jnp.float32(1.0))


def _mega_body(a_ref, base_ref, w1_ref, sens_ref,
               w2_ref, w3_ref, g1_ref, be1_ref, g2_ref, be2_ref,
               g3_ref, be3_ref, out_ref,
               h_ref, y_ref, s_ref, sum_ref, sq_ref, dv_ref):
    step = pl.program_id(0)
    phase = step // G
    i = step % G

    def dv_blk():
        return dv_ref[pl.ds(i * RB, RB)]

    def accum_stats(y):
        prev_s = jnp.where(i == 0, jnp.float32(0.0), sum_ref[...])
        prev_q = jnp.where(i == 0, jnp.float32(0.0), sq_ref[...])
        sum_ref[...] = prev_s + jnp.sum(y, axis=0, keepdims=True)
        sq_ref[...] = prev_q + jnp.sum(y * y, axis=0, keepdims=True)

    @pl.when(phase == 0)
    def _p0():
        a_blk = a_ref[pl.ds(i * RB, RB), :]
        dv = lax.rsqrt(jnp.sum(a_blk, axis=1))
        dv_ref[pl.ds(i * RB, RB)] = dv
        h = jnp.dot(base_ref[...], w1_ref[...], preferred_element_type=F32)
        h_ref[pl.ds(i * RB, RB), :HID] = h * dv[:, None]
        @pl.when(i == G - 1)
        def _():
            sel = lax.broadcasted_iota(jnp.int32, (1, RB), 1) == (
                N - 2 - (G - 1) * RB)
            d2046 = jnp.sum(jnp.where(sel, dv[None, :], jnp.float32(0.0)))
            s_ref[...] = jnp.dot(sens_ref[...], w1_ref[...],
                                 preferred_element_type=F32) * d2046

    @pl.when(phase == 1)
    def _p1():
        dv = dv_blk()
        a_blk = a_ref[pl.ds(i * RB, RB), :]
        z = jnp.dot(a_blk, h_ref[:, :HID],
                    preferred_element_type=F32) * dv[:, None]
        u = a_blk[:, N - 2:N - 1] * dv[:, None]
        s = s_ref[...]
        for b in range(B):
            y_ref[pl.ds(i * RB, RB), b * HID:(b + 1) * HID] = (
                z + u * s[b:b + 1, :])
        accum_stats(y_ref[pl.ds(i * RB, RB), :])

    def bn_elu_mm(w_ref_n, g_r, be_r):
        scale, shift = _bn_scale_shift2(sum_ref, sq_ref, g_r, be_r)
        dv = dv_blk()
        y = y_ref[pl.ds(i * RB, RB), :]
        w = w_ref_n[...]
        for b in range(B):
            yb = (y[:, b * HID:(b + 1) * HID] * scale[None, :]
                  + shift[None, :])
            xb = jnp.where(yb > 0, yb, jnp.exp(yb) - jnp.float32(1.0))
            h_ref[pl.ds(i * RB, RB), b * HID:(b + 1) * HID] = jnp.dot(
                xb, w, preferred_element_type=F32) * dv[:, None]

    def agg():
        dv = dv_blk()
        y = jnp.dot(a_ref[pl.ds(i * RB, RB), :], h_ref[...],
                    preferred_element_type=F32) * dv[:, None]
        y_ref[pl.ds(i * RB, RB), :] = y
        accum_stats(y)

    @pl.when(phase == 2)
    def _p2():
        bn_elu_mm(w2_ref, g1_ref, be1_ref)

    @pl.when(phase == 3)
    def _p3():
        agg()

    @pl.when(phase == 4)
    def _p4():
        bn_elu_mm(w3_ref, g2_ref, be2_ref)

    @pl.when(phase == 5)
    def _p5():
        agg()

    @pl.when(phase == 6)
    def _p6():
        scale, shift = _bn_scale_shift2(sum_ref, sq_ref, g3_ref, be3_ref)
        y = y_ref[pl.ds(i * RB, RB), :]
        for b in range(B):
            yb = (y[:, b * HID:(b + 1) * HID] * scale[None, :]
                  + shift[None, :])
            out_ref[:, b * HID:(b + 1) * HID] = jnp.where(
                yb > 0, yb, jnp.exp(yb) - jnp.float32(1.0))


def _bn_scale_shift2(sum_ref, sq_ref, g_ref, be_ref):
    cs = sum_ref[...].reshape(B, HID)
    cq = sq_ref[...].reshape(B, HID)
    inv_n = jnp.float32(1.0 / (B * N))
    mu = jnp.sum(cs, axis=0) * inv_n
    ex2 = jnp.sum(cq, axis=0) * inv_n
    var = ex2 - mu * mu
    scale = lax.rsqrt(var + BN_EPS) * g_ref[...]
    shift = be_ref[...] - mu * scale
    return scale, shift


def _mega(A, base_p, W1, sensor, W2, W3, g1, be1, g2, be2, g3, be3):
    c0 = lambda s: (0, 0)
    c1 = lambda s: (0,)
    return pl.pallas_call(
        _mega_body,
        grid=(7 * G,),
        in_specs=[
            pl.BlockSpec((N, N), c0),
            pl.BlockSpec((RB, EMBED), lambda s: (jnp.minimum(s, G - 1), 0)),
            pl.BlockSpec((EMBED, HID), c0),
            pl.BlockSpec((B, EMBED), c0),
            pl.BlockSpec((HID, HID), c0),
            pl.BlockSpec((HID, HID), c0),
            pl.BlockSpec((HID,), c1),
            pl.BlockSpec((HID,), c1),
            pl.BlockSpec((HID,), c1),
            pl.BlockSpec((HID,), c1),
            pl.BlockSpec((HID,), c1),
            pl.BlockSpec((HID,), c1),
        ],
        out_specs=pl.BlockSpec(
            (RB, B * HID), lambda s: (jnp.clip(s - 6 * G, 0, G - 1), 0)),
        out_shape=jax.ShapeDtypeStruct((N, B * HID), F32),
        scratch_shapes=[
            pltpu.VMEM((N, B * HID), F32),
            pltpu.VMEM((N, B * HID), F32),
            pltpu.VMEM((B, HID), F32),
            pltpu.VMEM((1, B * HID), F32),
            pltpu.VMEM((1, B * HID), F32),
            pltpu.VMEM((N,), F32),
        ],
        compiler_params=_TCPARAMS,
    )(A, base_p, W1, sensor, W2, W3, g1, be1, g2, be2, g3, be3)


def _k1_body(base_ref, w_ref, sens_ref, degb_ref, degf_ref, h_ref, s_ref):
    dv = _dinv(degb_ref[...])
    h = jnp.dot(base_ref[...], w_ref[...], preferred_element_type=F32)
    h_ref[...] = h * dv[:, None]
    @pl.when(pl.program_id(0) == 0)
    def _():
        dvf = _dinv(degf_ref[...])
        sel = lax.broadcasted_iota(jnp.int32, (1, N), 1) == (N - 2)
        d2046 = jnp.sum(jnp.where(sel, dvf[None, :], jnp.float32(0.0)))
        s_ref[...] = jnp.dot(sens_ref[...], w_ref[...],
                             preferred_element_type=F32) * d2046


def _layer1_h(base_p, W1, sensor, deg):
    return pl.pallas_call(
        _k1_body,
        grid=(G,),
        in_specs=[
            pl.BlockSpec((RB, EMBED), lambda i: (i, 0)),
            pl.BlockSpec((EMBED, HID), lambda i: (0, 0)),
            pl.BlockSpec((B, EMBED), lambda i: (0, 0)),
            pl.BlockSpec((RB,), lambda i: (i,)),
            pl.BlockSpec((N,), lambda i: (0,)),
        ],
        out_specs=[
            pl.BlockSpec((RB, HID), lambda i: (i, 0)),
            pl.BlockSpec((B, HID), lambda i: (0, 0)),
        ],
        out_shape=[
            jax.ShapeDtypeStruct((N, HID), F32),
            jax.ShapeDtypeStruct((B, HID), F32),
        ],
        compiler_params=_TCPARAMS,
    )(base_p, W1, sensor, deg, deg)


def _k2l1_body(a_ref, h_ref, s_ref, degb_ref, y_ref, sum_ref, sq_ref):
    dv = _dinv(degb_ref[...])
    z = jnp.dot(a_ref[...], h_ref[...], preferred_element_type=F32)
    u = a_ref[:, N - 2:N - 1] * dv[:, None]
    z = z * dv[:, None]
    s = s_ref[...]
    for b in range(B):
        y_ref[:, b * HID:(b + 1) * HID] = z + u * s[b:b + 1, :]
    y = y_ref[...]
    sum_ref[0, ...] = jnp.sum(y, axis=0, keepdims=True)
    sq_ref[0, ...] = jnp.sum(y * y, axis=0, keepdims=True)


def _k2_body(a_ref, h_ref, degb_ref, y_ref, sum_ref, sq_ref):
    dv = _dinv(degb_ref[...])
    y = jnp.dot(a_ref[...], h_ref[...], preferred_element_type=F32)
    y = y * dv[:, None]
    y_ref[...] = y
    sum_ref[0, ...] = jnp.sum(y, axis=0, keepdims=True)
    sq_ref[0, ...] = jnp.sum(y * y, axis=0, keepdims=True)


def _agg_l1(A, H1, S, deg):
    return pl.pallas_call(
        _k2l1_body,
        grid=(G,),
        in_specs=[
            pl.BlockSpec((RB, N), lambda i: (i, 0)),
            pl.BlockSpec((N, HID), lambda i: (0, 0)),
            pl.BlockSpec((B, HID), lambda i: (0, 0)),
            pl.BlockSpec((RB,), lambda i: (i,)),
        ],
        out_specs=[
            pl.BlockSpec((RB, B * HID), lambda i: (i, 0)),
            pl.BlockSpec((1, 1, B * HID), lambda i: (i, 0, 0)),
            pl.BlockSpec((1, 1, B * HID), lambda i: (i, 0, 0)),
        ],
        out_shape=[
            jax.ShapeDtypeStruct((N, B * HID), F32),
            jax.ShapeDtypeStruct((G, 1, B * HID), F32),
            jax.ShapeDtypeStruct((G, 1, B * HID), F32),
        ],
        compiler_params=_TCPARAMS,
    )(A, H1, S, deg)


def _agg(A, H, deg):
    return pl.pallas_call(
        _k2_body,
        grid=(G,),
        in_specs=[
            pl.BlockSpec((RB, N), lambda i: (i, 0)),
            pl.BlockSpec((N, B * HID), lambda i: (0, 0)),
            pl.BlockSpec((RB,), lambda i: (i,)),
        ],
        out_specs=[
            pl.BlockSpec((RB, B * HID), lambda i: (i, 0)),
            pl.BlockSpec((1, 1, B * HID), lambda i: (i, 0, 0)),
            pl.BlockSpec((1, 1, B * HID), lambda i: (i, 0, 0)),
        ],
        out_shape=[
            jax.ShapeDtypeStruct((N, B * HID), F32),
            jax.ShapeDtypeStruct((G, 1, B * HID), F32),
            jax.ShapeDtypeStruct((G, 1, B * HID), F32),
        ],
        compiler_params=_TCPARAMS,
    )(A, H, deg)


def _bn_scale_shift(sum_ref, sq_ref, g_ref, be_ref):
    cs = jnp.sum(sum_ref[...], axis=(0, 1)).reshape(B, HID)
    cq = jnp.sum(sq_ref[...], axis=(0, 1)).reshape(B, HID)
    inv_n = jnp.float32(1.0 / (B * N))
    mu = jnp.sum(cs, axis=0) * inv_n
    ex2 = jnp.sum(cq, axis=0) * inv_n
    var = ex2 - mu * mu
    scale = lax.rsqrt(var + BN_EPS) * g_ref[...]
    shift = be_ref[...] - mu * scale
    return scale, shift


def _k3_body(y_ref, sum_ref, sq_ref, g_ref, be_ref, w_ref, degb_ref, h_ref):
    scale, shift = _bn_scale_shift(sum_ref, sq_ref, g_ref, be_ref)
    dv = _dinv(degb_ref[...])
    y = y_ref[...]
    w = w_ref[...]
    for b in range(B):
        yb = y[:, b * HID:(b + 1) * HID] * scale[None, :] + shift[None, :]
        xb = jnp.where(yb > 0, yb, jnp.exp(yb) - jnp.float32(1.0))
        h_ref[:, b * HID:(b + 1) * HID] = jnp.dot(
            xb, w, preferred_element_type=F32) * dv[:, None]


def _k3f_body(y_ref, sum_ref, sq_ref, g_ref, be_ref, x_ref):
    scale, shift = _bn_scale_shift(sum_ref, sq_ref, g_ref, be_ref)
    y = y_ref[...]
    for b in range(B):
        yb = y[:, b * HID:(b + 1) * HID] * scale[None, :] + shift[None, :]
        x_ref[:, b * HID:(b + 1) * HID] = jnp.where(
            yb > 0, yb, jnp.exp(yb) - jnp.float32(1.0))


def _bn_elu_mm(Y, ssum, ssq, g, be, Wn, deg):
    return pl.pallas_call(
        _k3_body,
        grid=(G,),
        in_specs=[
            pl.BlockSpec((RB, B * HID), lambda i: (i, 0)),
            pl.BlockSpec((G, 1, B * HID), lambda i: (0, 0, 0)),
            pl.BlockSpec((G, 1, B * HID), lambda i: (0, 0, 0)),
            pl.BlockSpec((HID,), lambda i: (0,)),
            pl.BlockSpec((HID,), lambda i: (0,)),
            pl.BlockSpec((HID, HID), lambda i: (0, 0)),
            pl.BlockSpec((RB,), lambda i: (i,)),
        ],
        out_specs=pl.BlockSpec((RB, B * HID), lambda i: (i, 0)),
        out_shape=jax.ShapeDtypeStruct((N, B * HID), F32),
        compiler_params=_TCPARAMS,
    )(Y, ssum, ssq, g, be, Wn, deg)


def _bn_elu(Y, ssum, ssq, g, be):
    return pl.pallas_call(
        _k3f_body,
        grid=(G,),
        in_specs=[
            pl.BlockSpec((RB, B * HID), lambda i: (i, 0)),
            pl.BlockSpec((G, 1, B * HID), lambda i: (0, 0, 0)),
            pl.BlockSpec((G, 1, B * HID), lambda i: (0, 0, 0)),
            pl.BlockSpec((HID,), lambda i: (0,)),
            pl.BlockSpec((HID,), lambda i: (0,)),
        ],
        out_specs=pl.BlockSpec((RB, B * HID), lambda i: (i, 0)),
        out_shape=jax.ShapeDtypeStruct((N, B * HID), F32),
        compiler_params=_TCPARAMS,
    )(Y, ssum, ssq, g, be)


def kernel(sensor_batch, base_vertices, edge_index,
           W1, b1, g1, be1, W2, b2, g2, be2, W3, b3, g3, be3):
    A = _sc_build_adjacency(edge_index[0], edge_index[1]).reshape(N, N)

    base_p = jnp.concatenate(
        [base_vertices, jnp.zeros((2, EMBED), F32)], axis=0)
    X3 = _mega(A, base_p, W1, sensor_batch,
               W2, W3, g1, be1, g2, be2, g3, be3)

    return X3.reshape(N, B, HID).transpose(1, 0, 2)


# speedup vs baseline: 98.2711x; 1.0346x over previous
import functools

import jax
import jax.numpy as jnp
from jax import lax
from jax.experimental import pallas as pl
from jax.experimental.pallas import tpu as pltpu
from jax.experimental.pallas import tpu_sc as plsc

F32 = jnp.float32
EMBED = 1024
HID = 128
N = 2048
E = 32768
B = 8
BN_EPS = 1e-5

NC = 2
NS = 16


PASSES = ((0, 48), (1536, 16))
ECH = 8192


def _sc_build_adjacency(src1d, dst1d):
    mesh = plsc.VectorSubcoreMesh(core_axis_name="c", subcore_axis_name="s")

    @functools.partial(
        pl.kernel,
        out_type=jax.ShapeDtypeStruct((N * N,), F32),
        mesh=mesh,
        scratch_types=[
            pltpu.VMEM((ECH,), jnp.int32),
            pltpu.VMEM((ECH,), jnp.int32),
            pltpu.VMEM((PASSES[0][1] * N,), F32),
        ],
        compiler_params=pltpu.CompilerParams(needs_layout_passes=False),
    )
    def build(src_h, dst_h, a_h, src_v, dst_v, acc_v):
        cid = lax.axis_index("c")
        sid = lax.axis_index("s")
        wid = cid * NS + sid

        for base_row, rpt in PASSES:
            lo = base_row + wid * rpt
            nw = rpt * N

            def zfill(i, _):
                acc_v[pl.ds(i * 16, 16)] = jnp.zeros((16,), F32)
                return 0
            lax.fori_loop(0, nw // 16, zfill, 0, unroll=8)

            ones = jnp.full((16,), 1.0, F32)
            tmask = jnp.full((16,), True)
            for q in range(rpt // 16):
                rel = q * 16 + lax.iota(jnp.int32, 16)
                plsc.addupdate_scatter(
                    acc_v, [rel * (N + 1) + lo], ones, mask=tmask)

            for ch in range(E // ECH):
                pltpu.sync_copy(src_h.at[pl.ds(ch * ECH, ECH)], src_v)
                pltpu.sync_copy(dst_h.at[pl.ds(ch * ECH, ECH)], dst_v)

                def grp(g, _):
                    sl = pl.ds(g * 16, 16)
                    s = src_v[sl]
                    rel = dst_v[sl] - lo
                    m = (rel >= 0) & (rel < rpt)
                    idx = jnp.where(m, rel * N + s, 0)
                    plsc.addupdate_scatter(
                        acc_v, [idx], jnp.full((16,), 1.0, F32), mask=m)
                    return 0
                lax.fori_loop(0, ECH // 16, grp, 0, unroll=8)

            pltpu.sync_copy(acc_v.at[pl.ds(0, nw)],
                            a_h.at[pl.ds(lo * N, nw)])

    return build(src1d, dst1d)


_TCPARAMS = pltpu.CompilerParams(dimension_semantics=("arbitrary",))
RB = 256
G = N // RB


def _dinv(deg):
    return lax.rsqrt(deg + jnp.float32(1.0))


def _mega_body(a_ref, base_ref, w1_ref, sens_ref,
               w2_ref, w3_ref, g1_ref, be1_ref, g2_ref, be2_ref,
               g3_ref, be3_ref, out_ref,
               h_ref, y_ref, s_ref, sum_ref, sq_ref, dv_ref, ab_ref):
    step = pl.program_id(0)
    phase = step // G
    i = step % G

    def dv_blk():
        return dv_ref[pl.ds(i * RB, RB)]

    def accum_stats(y):
        prev_s = jnp.where(i == 0, jnp.float32(0.0), sum_ref[...])
        prev_q = jnp.where(i == 0, jnp.float32(0.0), sq_ref[...])
        sum_ref[...] = prev_s + jnp.sum(y, axis=0, keepdims=True)
        sq_ref[...] = prev_q + jnp.sum(y * y, axis=0, keepdims=True)

    @pl.when(phase == 0)
    def _p0():
        a_blk = a_ref[pl.ds(i * RB, RB), :]
        dv = lax.rsqrt(jnp.sum(a_blk, axis=1))
        dv_ref[pl.ds(i * RB, RB)] = dv
        ab_ref[pl.ds(i * RB, RB), :] = a_blk.astype(jnp.bfloat16)
        h = jnp.dot(base_ref[...], w1_ref[...], preferred_element_type=F32)
        h_ref[pl.ds(i * RB, RB), :HID] = (h * dv[:, None]).astype(jnp.bfloat16)
        @pl.when(i == G - 1)
        def _():
            sel = lax.broadcasted_iota(jnp.int32, (1, RB), 1) == (
                N - 2 - (G - 1) * RB)
            d2046 = jnp.sum(jnp.where(sel, dv[None, :], jnp.float32(0.0)))
            s_ref[...] = jnp.dot(sens_ref[...], w1_ref[...],
                                 preferred_element_type=F32) * d2046

    @pl.when(phase == 1)
    def _p1():
        dv = dv_blk()
        z = jnp.dot(ab_ref[pl.ds(i * RB, RB), :], h_ref[:, :HID],
                    preferred_element_type=F32) * dv[:, None]
        u = a_ref[pl.ds(i * RB, RB), N - 2:N - 1] * dv[:, None]
        s = s_ref[...]
        for b in range(B):
            y_ref[pl.ds(i * RB, RB), b * HID:(b + 1) * HID] = (
                z + u * s[b:b + 1, :])
        accum_stats(y_ref[pl.ds(i * RB, RB), :])

    def bn_elu_mm(w_ref_n, g_r, be_r):
        scale, shift = _bn_scale_shift2(sum_ref, sq_ref, g_r, be_r)
        dv = dv_blk()
        y = y_ref[pl.ds(i * RB, RB), :]
        w = w_ref_n[...]
        for b in range(B):
            yb = (y[:, b * HID:(b + 1) * HID] * scale[None, :]
                  + shift[None, :])
            xb = jnp.where(yb > 0, yb, jnp.exp(yb) - jnp.float32(1.0))
            h_ref[pl.ds(i * RB, RB), b * HID:(b + 1) * HID] = (
                jnp.dot(xb, w, preferred_element_type=F32)
                * dv[:, None]).astype(jnp.bfloat16)

    def agg():
        dv = dv_blk()
        y = jnp.dot(ab_ref[pl.ds(i * RB, RB), :], h_ref[...],
                    preferred_element_type=F32) * dv[:, None]
        y_ref[pl.ds(i * RB, RB), :] = y
        accum_stats(y)

    @pl.when(phase == 2)
    def _p2():
        bn_elu_mm(w2_ref, g1_ref, be1_ref)

    @pl.when(phase == 3)
    def _p3():
        agg()

    @pl.when(phase == 4)
    def _p4():
        bn_elu_mm(w3_ref, g2_ref, be2_ref)

    @pl.when(phase == 5)
    def _p5():
        agg()

    @pl.when(phase == 6)
    def _p6():
        scale, shift = _bn_scale_shift2(sum_ref, sq_ref, g3_ref, be3_ref)
        y = y_ref[pl.ds(i * RB, RB), :]
        for b in range(B):
            yb = (y[:, b * HID:(b + 1) * HID] * scale[None, :]
                  + shift[None, :])
            out_ref[:, b * HID:(b + 1) * HID] = jnp.where(
                yb > 0, yb, jnp.exp(yb) - jnp.float32(1.0))


def _bn_scale_shift2(sum_ref, sq_ref, g_ref, be_ref):
    cs = sum_ref[...].reshape(B, HID)
    cq = sq_ref[...].reshape(B, HID)
    inv_n = jnp.float32(1.0 / (B * N))
    mu = jnp.sum(cs, axis=0) * inv_n
    ex2 = jnp.sum(cq, axis=0) * inv_n
    var = ex2 - mu * mu
    scale = lax.rsqrt(var + BN_EPS) * g_ref[...]
    shift = be_ref[...] - mu * scale
    return scale, shift


def _mega(A, base_p, W1, sensor, W2, W3, g1, be1, g2, be2, g3, be3):
    c0 = lambda s: (0, 0)
    c1 = lambda s: (0,)
    return pl.pallas_call(
        _mega_body,
        grid=(7 * G,),
        in_specs=[
            pl.BlockSpec((N, N), c0),
            pl.BlockSpec((RB, EMBED), lambda s: (jnp.minimum(s, G - 1), 0)),
            pl.BlockSpec((EMBED, HID), c0),
            pl.BlockSpec((B, EMBED), c0),
            pl.BlockSpec((HID, HID), c0),
            pl.BlockSpec((HID, HID), c0),
            pl.BlockSpec((HID,), c1),
            pl.BlockSpec((HID,), c1),
            pl.BlockSpec((HID,), c1),
            pl.BlockSpec((HID,), c1),
            pl.BlockSpec((HID,), c1),
            pl.BlockSpec((HID,), c1),
        ],
        out_specs=pl.BlockSpec(
            (RB, B * HID), lambda s: (jnp.clip(s - 6 * G, 0, G - 1), 0)),
        out_shape=jax.ShapeDtypeStruct((N, B * HID), F32),
        scratch_shapes=[
            pltpu.VMEM((N, B * HID), jnp.bfloat16),
            pltpu.VMEM((N, B * HID), F32),
            pltpu.VMEM((B, HID), F32),
            pltpu.VMEM((1, B * HID), F32),
            pltpu.VMEM((1, B * HID), F32),
            pltpu.VMEM((N,), F32),
            pltpu.VMEM((N, N), jnp.bfloat16),
        ],
        compiler_params=_TCPARAMS,
    )(A, base_p, W1, sensor, W2, W3, g1, be1, g2, be2, g3, be3)


def _k1_body(base_ref, w_ref, sens_ref, degb_ref, degf_ref, h_ref, s_ref):
    dv = _dinv(degb_ref[...])
    h = jnp.dot(base_ref[...], w_ref[...], preferred_element_type=F32)
    h_ref[...] = h * dv[:, None]
    @pl.when(pl.program_id(0) == 0)
    def _():
        dvf = _dinv(degf_ref[...])
        sel = lax.broadcasted_iota(jnp.int32, (1, N), 1) == (N - 2)
        d2046 = jnp.sum(jnp.where(sel, dvf[None, :], jnp.float32(0.0)))
        s_ref[...] = jnp.dot(sens_ref[...], w_ref[...],
                             preferred_element_type=F32) * d2046


def _layer1_h(base_p, W1, sensor, deg):
    return pl.pallas_call(
        _k1_body,
        grid=(G,),
        in_specs=[
            pl.BlockSpec((RB, EMBED), lambda i: (i, 0)),
            pl.BlockSpec((EMBED, HID), lambda i: (0, 0)),
            pl.BlockSpec((B, EMBED), lambda i: (0, 0)),
            pl.BlockSpec((RB,), lambda i: (i,)),
            pl.BlockSpec((N,), lambda i: (0,)),
        ],
        out_specs=[
            pl.BlockSpec((RB, HID), lambda i: (i, 0)),
            pl.BlockSpec((B, HID), lambda i: (0, 0)),
        ],
        out_shape=[
            jax.ShapeDtypeStruct((N, HID), F32),
            jax.ShapeDtypeStruct((B, HID), F32),
        ],
        compiler_params=_TCPARAMS,
    )(base_p, W1, sensor, deg, deg)


def _k2l1_body(a_ref, h_ref, s_ref, degb_ref, y_ref, sum_ref, sq_ref):
    dv = _dinv(degb_ref[...])
    z = jnp.dot(a_ref[...], h_ref[...], preferred_element_type=F32)
    u = a_ref[:, N - 2:N - 1] * dv[:, None]
    z = z * dv[:, None]
    s = s_ref[...]
    for b in range(B):
        y_ref[:, b * HID:(b + 1) * HID] = z + u * s[b:b + 1, :]
    y = y_ref[...]
    sum_ref[0, ...] = jnp.sum(y, axis=0, keepdims=True)
    sq_ref[0, ...] = jnp.sum(y * y, axis=0, keepdims=True)


def _k2_body(a_ref, h_ref, degb_ref, y_ref, sum_ref, sq_ref):
    dv = _dinv(degb_ref[...])
    y = jnp.dot(a_ref[...], h_ref[...], preferred_element_type=F32)
    y = y * dv[:, None]
    y_ref[...] = y
    sum_ref[0, ...] = jnp.sum(y, axis=0, keepdims=True)
    sq_ref[0, ...] = jnp.sum(y * y, axis=0, keepdims=True)


def _agg_l1(A, H1, S, deg):
    return pl.pallas_call(
        _k2l1_body,
        grid=(G,),
        in_specs=[
            pl.BlockSpec((RB, N), lambda i: (i, 0)),
            pl.BlockSpec((N, HID), lambda i: (0, 0)),
            pl.BlockSpec((B, HID), lambda i: (0, 0)),
            pl.BlockSpec((RB,), lambda i: (i,)),
        ],
        out_specs=[
            pl.BlockSpec((RB, B * HID), lambda i: (i, 0)),
            pl.BlockSpec((1, 1, B * HID), lambda i: (i, 0, 0)),
            pl.BlockSpec((1, 1, B * HID), lambda i: (i, 0, 0)),
        ],
        out_shape=[
            jax.ShapeDtypeStruct((N, B * HID), F32),
            jax.ShapeDtypeStruct((G, 1, B * HID), F32),
            jax.ShapeDtypeStruct((G, 1, B * HID), F32),
        ],
        compiler_params=_TCPARAMS,
    )(A, H1, S, deg)


def _agg(A, H, deg):
    return pl.pallas_call(
        _k2_body,
        grid=(G,),
        in_specs=[
            pl.BlockSpec((RB, N), lambda i: (i, 0)),
            pl.BlockSpec((N, B * HID), lambda i: (0, 0)),
            pl.BlockSpec((RB,), lambda i: (i,)),
        ],
        out_specs=[
            pl.BlockSpec((RB, B * HID), lambda i: (i, 0)),
            pl.BlockSpec((1, 1, B * HID), lambda i: (i, 0, 0)),
            pl.BlockSpec((1, 1, B * HID), lambda i: (i, 0, 0)),
        ],
        out_shape=[
            jax.ShapeDtypeStruct((N, B * HID), F32),
            jax.ShapeDtypeStruct((G, 1, B * HID), F32),
            jax.ShapeDtypeStruct((G, 1, B * HID), F32),
        ],
        compiler_params=_TCPARAMS,
    )(A, H, deg)


def _bn_scale_shift(sum_ref, sq_ref, g_ref, be_ref):
    cs = jnp.sum(sum_ref[...], axis=(0, 1)).reshape(B, HID)
    cq = jnp.sum(sq_ref[...], axis=(0, 1)).reshape(B, HID)
    inv_n = jnp.float32(1.0 / (B * N))
    mu = jnp.sum(cs, axis=0) * inv_n
    ex2 = jnp.sum(cq, axis=0) * inv_n
    var = ex2 - mu * mu
    scale = lax.rsqrt(var + BN_EPS) * g_ref[...]
    shift = be_ref[...] - mu * scale
    return scale, shift


def _k3_body(y_ref, sum_ref, sq_ref, g_ref, be_ref, w_ref, degb_ref, h_ref):
    scale, shift = _bn_scale_shift(sum_ref, sq_ref, g_ref, be_ref)
    dv = _dinv(degb_ref[...])
    y = y_ref[...]
    w = w_ref[...]
    for b in range(B):
        yb = y[:, b * HID:(b + 1) * HID] * scale[None, :] + shift[None, :]
        xb = jnp.where(yb > 0, yb, jnp.exp(yb) - jnp.float32(1.0))
        h_ref[:, b * HID:(b + 1) * HID] = jnp.dot(
            xb, w, preferred_element_type=F32) * dv[:, None]


def _k3f_body(y_ref, sum_ref, sq_ref, g_ref, be_ref, x_ref):
    scale, shift = _bn_scale_shift(sum_ref, sq_ref, g_ref, be_ref)
    y = y_ref[...]
    for b in range(B):
        yb = y[:, b * HID:(b + 1) * HID] * scale[None, :] + shift[None, :]
        x_ref[:, b * HID:(b + 1) * HID] = jnp.where(
            yb > 0, yb, jnp.exp(yb) - jnp.float32(1.0))


def _bn_elu_mm(Y, ssum, ssq, g, be, Wn, deg):
    return pl.pallas_call(
        _k3_body,
        grid=(G,),
        in_specs=[
            pl.BlockSpec((RB, B * HID), lambda i: (i, 0)),
            pl.BlockSpec((G, 1, B * HID), lambda i: (0, 0, 0)),
            pl.BlockSpec((G, 1, B * HID), lambda i: (0, 0, 0)),
            pl.BlockSpec((HID,), lambda i: (0,)),
            pl.BlockSpec((HID,), lambda i: (0,)),
            pl.BlockSpec((HID, HID), lambda i: (0, 0)),
            pl.BlockSpec((RB,), lambda i: (i,)),
        ],
        out_specs=pl.BlockSpec((RB, B * HID), lambda i: (i, 0)),
        out_shape=jax.ShapeDtypeStruct((N, B * HID), F32),
        compiler_params=_TCPARAMS,
    )(Y, ssum, ssq, g, be, Wn, deg)


def _bn_elu(Y, ssum, ssq, g, be):
    return pl.pallas_call(
        _k3f_body,
        grid=(G,),
        in_specs=[
            pl.BlockSpec((RB, B * HID), lambda i: (i, 0)),
            pl.BlockSpec((G, 1, B * HID), lambda i: (0, 0, 0)),
            pl.BlockSpec((G, 1, B * HID), lambda i: (0, 0, 0)),
            pl.BlockSpec((HID,), lambda i: (0,)),
            pl.BlockSpec((HID,), lambda i: (0,)),
        ],
        out_specs=pl.BlockSpec((RB, B * HID), lambda i: (i, 0)),
        out_shape=jax.ShapeDtypeStruct((N, B * HID), F32),
        compiler_params=_TCPARAMS,
    )(Y, ssum, ssq, g, be)


def kernel(sensor_batch, base_vertices, edge_index,
           W1, b1, g1, be1, W2, b2, g2, be2, W3, b3, g3, be3):
    A = _sc_build_adjacency(edge_index[0], edge_index[1]).reshape(N, N)

    base_p = jnp.concatenate(
        [base_vertices, jnp.zeros((2, EMBED), F32)], axis=0)
    X3 = _mega(A, base_p, W1, sensor_batch,
               W2, W3, g1, be1, g2, be2, g3, be3)

    return X3.reshape(N, B, HID).transpose(1, 0, 2)


# speedup vs baseline: 114.8239x; 1.1684x over previous
import functools

import jax
import jax.numpy as jnp
from jax import lax
from jax.experimental import pallas as pl
from jax.experimental.pallas import tpu as pltpu
from jax.experimental.pallas import tpu_sc as plsc

F32 = jnp.float32
EMBED = 1024
HID = 128
N = 2048
E = 32768
B = 8
BN_EPS = 1e-5

NC = 2
NS = 16


PASSES = ((0, 48), (1536, 16))
ECH = 8192


def _sc_build_adjacency(src1d, dst1d):
    mesh = plsc.VectorSubcoreMesh(core_axis_name="c", subcore_axis_name="s")

    @functools.partial(
        pl.kernel,
        out_type=jax.ShapeDtypeStruct((N * N,), F32),
        mesh=mesh,
        scratch_types=[
            pltpu.VMEM((2, ECH), jnp.int32),
            pltpu.VMEM((2, ECH), jnp.int32),
            pltpu.VMEM((PASSES[0][1] * N,), F32),
            pltpu.SemaphoreType.DMA,
            pltpu.SemaphoreType.DMA,
        ],
        compiler_params=pltpu.CompilerParams(needs_layout_passes=False),
    )
    def build(src_h, dst_h, a_h, src_v, dst_v, acc_v, sem0, sem1):
        cid = lax.axis_index("c")
        sid = lax.axis_index("s")
        wid = cid * NS + sid
        sems = (sem0, sem1)
        NCH = E // ECH

        def start(ch, buf):
            pltpu.async_copy(src_h.at[pl.ds(ch * ECH, ECH)],
                             src_v.at[buf], sems[buf])
            pltpu.async_copy(dst_h.at[pl.ds(ch * ECH, ECH)],
                             dst_v.at[buf], sems[buf])

        def drain(buf):
            pltpu.make_async_copy(src_h.at[pl.ds(0, ECH)],
                                  src_v.at[buf], sems[buf]).wait()
            pltpu.make_async_copy(dst_h.at[pl.ds(0, ECH)],
                                  dst_v.at[buf], sems[buf]).wait()

        start(0, 0)
        for pi, (base_row, rpt) in enumerate(PASSES):
            lo = base_row + wid * rpt
            nw = rpt * N

            def zfill(i, _):
                acc_v[pl.ds(i * 16, 16)] = jnp.zeros((16,), F32)
                return 0
            lax.fori_loop(0, nw // 16, zfill, 0, unroll=8)

            ones = jnp.full((16,), 1.0, F32)
            tmask = jnp.full((16,), True)
            for q in range(rpt // 16):
                rel = q * 16 + lax.iota(jnp.int32, 16)
                plsc.addupdate_scatter(
                    acc_v, [rel * (N + 1) + lo], ones, mask=tmask)

            for ch in range(NCH):
                buf = ch % 2
                drain(buf)
                nxt = ch + 1
                if nxt == NCH and pi + 1 < len(PASSES):
                    start(0, nxt % 2)
                elif nxt < NCH:
                    start(nxt, nxt % 2)

                def grp(g, _, buf=buf):
                    sl = pl.ds(g * 16, 16)
                    s = src_v[buf, sl]
                    rel = dst_v[buf, sl] - lo
                    m = (rel >= 0) & (rel < rpt)
                    idx = jnp.where(m, rel * N + s, 0)
                    plsc.addupdate_scatter(
                        acc_v, [idx], jnp.full((16,), 1.0, F32), mask=m)
                    return 0
                lax.fori_loop(0, ECH // 16, grp, 0, unroll=8)

            pltpu.sync_copy(acc_v.at[pl.ds(0, nw)],
                            a_h.at[pl.ds(lo * N, nw)])

    return build(src1d, dst1d)


_TCPARAMS = pltpu.CompilerParams(dimension_semantics=("arbitrary",))
RB = 512
G = N // RB


def _dinv(deg):
    return lax.rsqrt(deg + jnp.float32(1.0))


def _mega_body(a_ref, base_ref, w1_ref, sens_ref,
               w2_ref, w3_ref, g1_ref, be1_ref, g2_ref, be2_ref,
               g3_ref, be3_ref, out_ref,
               h_ref, y_ref, s_ref, sum_ref, sq_ref, dv_ref, ab_ref):
    step = pl.program_id(0)
    phase = step // G
    i = step % G

    def dv_blk():
        return dv_ref[pl.ds(i * RB, RB)]

    def accum_stats(y):
        prev_s = jnp.where(i == 0, jnp.float32(0.0), sum_ref[...])
        prev_q = jnp.where(i == 0, jnp.float32(0.0), sq_ref[...])
        sum_ref[...] = prev_s + jnp.sum(y, axis=0, keepdims=True)
        sq_ref[...] = prev_q + jnp.sum(y * y, axis=0, keepdims=True)

    @pl.when(phase == 0)
    def _p0():
        a_blk = a_ref[pl.ds(i * RB, RB), :]
        dv = lax.rsqrt(jnp.sum(a_blk, axis=1))
        dv_ref[pl.ds(i * RB, RB)] = dv
        ab_ref[pl.ds(i * RB, RB), :] = a_blk.astype(jnp.bfloat16)
        h = jnp.dot(base_ref[...], w1_ref[...], preferred_element_type=F32)
        h_ref[pl.ds(i * RB, RB), :HID] = (h * dv[:, None]).astype(jnp.bfloat16)
        @pl.when(i == G - 1)
        def _():
            sel = lax.broadcasted_iota(jnp.int32, (1, RB), 1) == (
                N - 2 - (G - 1) * RB)
            d2046 = jnp.sum(jnp.where(sel, dv[None, :], jnp.float32(0.0)))
            s_ref[...] = jnp.dot(sens_ref[...], w1_ref[...],
                                 preferred_element_type=F32) * d2046

    @pl.when(phase == 1)
    def _p1():
        dv = dv_blk()
        z = jnp.dot(ab_ref[pl.ds(i * RB, RB), :], h_ref[:, :HID],
                    preferred_element_type=F32) * dv[:, None]
        u = a_ref[pl.ds(i * RB, RB), N - 2:N - 1] * dv[:, None]
        s = s_ref[...]
        for b in range(B):
            y_ref[pl.ds(i * RB, RB), b * HID:(b + 1) * HID] = (
                z + u * s[b:b + 1, :])
        accum_stats(y_ref[pl.ds(i * RB, RB), :])

    def bn_elu_mm(w_ref_n, g_r, be_r):
        scale, shift = _bn_scale_shift2(sum_ref, sq_ref, g_r, be_r)
        dv = dv_blk()
        y = y_ref[pl.ds(i * RB, RB), :]
        w = w_ref_n[...]
        for b in range(B):
            yb = (y[:, b * HID:(b + 1) * HID] * scale[None, :]
                  + shift[None, :])
            xb = jnp.where(yb > 0, yb, jnp.exp(yb) - jnp.float32(1.0))
            h_ref[pl.ds(i * RB, RB), b * HID:(b + 1) * HID] = (
                jnp.dot(xb, w, preferred_element_type=F32)
                * dv[:, None]).astype(jnp.bfloat16)

    def agg():
        dv = dv_blk()
        y = jnp.dot(ab_ref[pl.ds(i * RB, RB), :], h_ref[...],
                    preferred_element_type=F32) * dv[:, None]
        y_ref[pl.ds(i * RB, RB), :] = y
        accum_stats(y)

    @pl.when(phase == 2)
    def _p2():
        bn_elu_mm(w2_ref, g1_ref, be1_ref)

    @pl.when(phase == 3)
    def _p3():
        agg()

    @pl.when(phase == 4)
    def _p4():
        bn_elu_mm(w3_ref, g2_ref, be2_ref)

    @pl.when(phase == 5)
    def _p5():
        agg()

    @pl.when(phase == 6)
    def _p6():
        scale, shift = _bn_scale_shift2(sum_ref, sq_ref, g3_ref, be3_ref)
        y = y_ref[pl.ds(i * RB, RB), :]
        for b in range(B):
            yb = (y[:, b * HID:(b + 1) * HID] * scale[None, :]
                  + shift[None, :])
            out_ref[:, b * HID:(b + 1) * HID] = jnp.where(
                yb > 0, yb, jnp.exp(yb) - jnp.float32(1.0))


def _bn_scale_shift2(sum_ref, sq_ref, g_ref, be_ref):
    cs = sum_ref[...].reshape(B, HID)
    cq = sq_ref[...].reshape(B, HID)
    inv_n = jnp.float32(1.0 / (B * N))
    mu = jnp.sum(cs, axis=0) * inv_n
    ex2 = jnp.sum(cq, axis=0) * inv_n
    var = ex2 - mu * mu
    scale = lax.rsqrt(var + BN_EPS) * g_ref[...]
    shift = be_ref[...] - mu * scale
    return scale, shift


def _mega(A, base_p, W1, sensor, W2, W3, g1, be1, g2, be2, g3, be3):
    c0 = lambda s: (0, 0)
    c1 = lambda s: (0,)
    return pl.pallas_call(
        _mega_body,
        grid=(7 * G,),
        in_specs=[
            pl.BlockSpec((N, N), c0),
            pl.BlockSpec((RB, EMBED), lambda s: (jnp.minimum(s, G - 1), 0)),
            pl.BlockSpec((EMBED, HID), c0),
            pl.BlockSpec((B, EMBED), c0),
            pl.BlockSpec((HID, HID), c0),
            pl.BlockSpec((HID, HID), c0),
            pl.BlockSpec((HID,), c1),
            pl.BlockSpec((HID,), c1),
            pl.BlockSpec((HID,), c1),
            pl.BlockSpec((HID,), c1),
            pl.BlockSpec((HID,), c1),
            pl.BlockSpec((HID,), c1),
        ],
        out_specs=pl.BlockSpec(
            (RB, B * HID), lambda s: (jnp.clip(s - 6 * G, 0, G - 1), 0)),
        out_shape=jax.ShapeDtypeStruct((N, B * HID), F32),
        scratch_shapes=[
            pltpu.VMEM((N, B * HID), jnp.bfloat16),
            pltpu.VMEM((N, B * HID), F32),
            pltpu.VMEM((B, HID), F32),
            pltpu.VMEM((1, B * HID), F32),
            pltpu.VMEM((1, B * HID), F32),
            pltpu.VMEM((N,), F32),
            pltpu.VMEM((N, N), jnp.bfloat16),
        ],
        compiler_params=_TCPARAMS,
    )(A, base_p, W1, sensor, W2, W3, g1, be1, g2, be2, g3, be3)


def _k1_body(base_ref, w_ref, sens_ref, degb_ref, degf_ref, h_ref, s_ref):
    dv = _dinv(degb_ref[...])
    h = jnp.dot(base_ref[...], w_ref[...], preferred_element_type=F32)
    h_ref[...] = h * dv[:, None]
    @pl.when(pl.program_id(0) == 0)
    def _():
        dvf = _dinv(degf_ref[...])
        sel = lax.broadcasted_iota(jnp.int32, (1, N), 1) == (N - 2)
        d2046 = jnp.sum(jnp.where(sel, dvf[None, :], jnp.float32(0.0)))
        s_ref[...] = jnp.dot(sens_ref[...], w_ref[...],
                             preferred_element_type=F32) * d2046


def _layer1_h(base_p, W1, sensor, deg):
    return pl.pallas_call(
        _k1_body,
        grid=(G,),
        in_specs=[
            pl.BlockSpec((RB, EMBED), lambda i: (i, 0)),
            pl.BlockSpec((EMBED, HID), lambda i: (0, 0)),
            pl.BlockSpec((B, EMBED), lambda i: (0, 0)),
            pl.BlockSpec((RB,), lambda i: (i,)),
            pl.BlockSpec((N,), lambda i: (0,)),
        ],
        out_specs=[
            pl.BlockSpec((RB, HID), lambda i: (i, 0)),
            pl.BlockSpec((B, HID), lambda i: (0, 0)),
        ],
        out_shape=[
            jax.ShapeDtypeStruct((N, HID), F32),
            jax.ShapeDtypeStruct((B, HID), F32),
        ],
        compiler_params=_TCPARAMS,
    )(base_p, W1, sensor, deg, deg)


def _k2l1_body(a_ref, h_ref, s_ref, degb_ref, y_ref, sum_ref, sq_ref):
    dv = _dinv(degb_ref[...])
    z = jnp.dot(a_ref[...], h_ref[...], preferred_element_type=F32)
    u = a_ref[:, N - 2:N - 1] * dv[:, None]
    z = z * dv[:, None]
    s = s_ref[...]
    for b in range(B):
        y_ref[:, b * HID:(b + 1) * HID] = z + u * s[b:b + 1, :]
    y = y_ref[...]
    sum_ref[0, ...] = jnp.sum(y, axis=0, keepdims=True)
    sq_ref[0, ...] = jnp.sum(y * y, axis=0, keepdims=True)


def _k2_body(a_ref, h_ref, degb_ref, y_ref, sum_ref, sq_ref):
    dv = _dinv(degb_ref[...])
    y = jnp.dot(a_ref[...], h_ref[...], preferred_element_type=F32)
    y = y * dv[:, None]
    y_ref[...] = y
    sum_ref[0, ...] = jnp.sum(y, axis=0, keepdims=True)
    sq_ref[0, ...] = jnp.sum(y * y, axis=0, keepdims=True)


def _agg_l1(A, H1, S, deg):
    return pl.pallas_call(
        _k2l1_body,
        grid=(G,),
        in_specs=[
            pl.BlockSpec((RB, N), lambda i: (i, 0)),
            pl.BlockSpec((N, HID), lambda i: (0, 0)),
            pl.BlockSpec((B, HID), lambda i: (0, 0)),
            pl.BlockSpec((RB,), lambda i: (i,)),
        ],
        out_specs=[
            pl.BlockSpec((RB, B * HID), lambda i: (i, 0)),
            pl.BlockSpec((1, 1, B * HID), lambda i: (i, 0, 0)),
            pl.BlockSpec((1, 1, B * HID), lambda i: (i, 0, 0)),
        ],
        out_shape=[
            jax.ShapeDtypeStruct((N, B * HID), F32),
            jax.ShapeDtypeStruct((G, 1, B * HID), F32),
            jax.ShapeDtypeStruct((G, 1, B * HID), F32),
        ],
        compiler_params=_TCPARAMS,
    )(A, H1, S, deg)


def _agg(A, H, deg):
    return pl.pallas_call(
        _k2_body,
        grid=(G,),
        in_specs=[
            pl.BlockSpec((RB, N), lambda i: (i, 0)),
            pl.BlockSpec((N, B * HID), lambda i: (0, 0)),
            pl.BlockSpec((RB,), lambda i: (i,)),
        ],
        out_specs=[
            pl.BlockSpec((RB, B * HID), lambda i: (i, 0)),
            pl.BlockSpec((1, 1, B * HID), lambda i: (i, 0, 0)),
            pl.BlockSpec((1, 1, B * HID), lambda i: (i, 0, 0)),
        ],
        out_shape=[
            jax.ShapeDtypeStruct((N, B * HID), F32),
            jax.ShapeDtypeStruct((G, 1, B * HID), F32),
            jax.ShapeDtypeStruct((G, 1, B * HID), F32),
        ],
        compiler_params=_TCPARAMS,
    )(A, H, deg)


def _bn_scale_shift(sum_ref, sq_ref, g_ref, be_ref):
    cs = jnp.sum(sum_ref[...], axis=(0, 1)).reshape(B, HID)
    cq = jnp.sum(sq_ref[...], axis=(0, 1)).reshape(B, HID)
    inv_n = jnp.float32(1.0 / (B * N))
    mu = jnp.sum(cs, axis=0) * inv_n
    ex2 = jnp.sum(cq, axis=0) * inv_n
    var = ex2 - mu * mu
    scale = lax.rsqrt(var + BN_EPS) * g_ref[...]
    shift = be_ref[...] - mu * scale
    return scale, shift


def _k3_body(y_ref, sum_ref, sq_ref, g_ref, be_ref, w_ref, degb_ref, h_ref):
    scale, shift = _bn_scale_shift(sum_ref, sq_ref, g_ref, be_ref)
    dv = _dinv(degb_ref[...])
    y = y_ref[...]
    w = w_ref[...]
    for b in range(B):
        yb = y[:, b * HID:(b + 1) * HID] * scale[None, :] + shift[None, :]
        xb = jnp.where(yb > 0, yb, jnp.exp(yb) - jnp.float32(1.0))
        h_ref[:, b * HID:(b + 1) * HID] = jnp.dot(
            xb, w, preferred_element_type=F32) * dv[:, None]


def _k3f_body(y_ref, sum_ref, sq_ref, g_ref, be_ref, x_ref):
    scale, shift = _bn_scale_shift(sum_ref, sq_ref, g_ref, be_ref)
    y = y_ref[...]
    for b in range(B):
        yb = y[:, b * HID:(b + 1) * HID] * scale[None, :] + shift[None, :]
        x_ref[:, b * HID:(b + 1) * HID] = jnp.where(
            yb > 0, yb, jnp.exp(yb) - jnp.float32(1.0))


def _bn_elu_mm(Y, ssum, ssq, g, be, Wn, deg):
    return pl.pallas_call(
        _k3_body,
        grid=(G,),
        in_specs=[
            pl.BlockSpec((RB, B * HID), lambda i: (i, 0)),
            pl.BlockSpec((G, 1, B * HID), lambda i: (0, 0, 0)),
            pl.BlockSpec((G, 1, B * HID), lambda i: (0, 0, 0)),
            pl.BlockSpec((HID,), lambda i: (0,)),
            pl.BlockSpec((HID,), lambda i: (0,)),
            pl.BlockSpec((HID, HID), lambda i: (0, 0)),
            pl.BlockSpec((RB,), lambda i: (i,)),
        ],
        out_specs=pl.BlockSpec((RB, B * HID), lambda i: (i, 0)),
        out_shape=jax.ShapeDtypeStruct((N, B * HID), F32),
        compiler_params=_TCPARAMS,
    )(Y, ssum, ssq, g, be, Wn, deg)


def _bn_elu(Y, ssum, ssq, g, be):
    return pl.pallas_call(
        _k3f_body,
        grid=(G,),
        in_specs=[
            pl.BlockSpec((RB, B * HID), lambda i: (i, 0)),
            pl.BlockSpec((G, 1, B * HID), lambda i: (0, 0, 0)),
            pl.BlockSpec((G, 1, B * HID), lambda i: (0, 0, 0)),
            pl.BlockSpec((HID,), lambda i: (0,)),
            pl.BlockSpec((HID,), lambda i: (0,)),
        ],
        out_specs=pl.BlockSpec((RB, B * HID), lambda i: (i, 0)),
        out_shape=jax.ShapeDtypeStruct((N, B * HID), F32),
        compiler_params=_TCPARAMS,
    )(Y, ssum, ssq, g, be)


def kernel(sensor_batch, base_vertices, edge_index,
           W1, b1, g1, be1, W2, b2, g2, be2, W3, b3, g3, be3):
    A = _sc_build_adjacency(edge_index[0], edge_index[1]).reshape(N, N)

    base_p = jnp.concatenate(
        [base_vertices, jnp.zeros((2, EMBED), F32)], axis=0)
    X3 = _mega(A, base_p, W1, sensor_batch,
               W2, W3, g1, be1, g2, be2, g3, be3)

    return X3.reshape(N, B, HID).transpose(1, 0, 2)


# speedup vs baseline: 117.8813x; 1.0266x over previous
import functools

import jax
import jax.numpy as jnp
from jax import lax
from jax.experimental import pallas as pl
from jax.experimental.pallas import tpu as pltpu
from jax.experimental.pallas import tpu_sc as plsc

F32 = jnp.float32
EMBED = 1024
HID = 128
N = 2048
E = 32768
B = 8
BN_EPS = 1e-5

NC = 2
NS = 16


PASSES = ((0, 48), (1536, 16))
ECH = 8192


def _sc_build_adjacency(src1d, dst1d):
    mesh = plsc.VectorSubcoreMesh(core_axis_name="c", subcore_axis_name="s")

    @functools.partial(
        pl.kernel,
        out_type=jax.ShapeDtypeStruct((N * N,), F32),
        mesh=mesh,
        scratch_types=[
            pltpu.VMEM((2, ECH), jnp.int32),
            pltpu.VMEM((2, ECH), jnp.int32),
            pltpu.VMEM((PASSES[0][1] * N,), F32),
            pltpu.SemaphoreType.DMA,
            pltpu.SemaphoreType.DMA,
        ],
        compiler_params=pltpu.CompilerParams(needs_layout_passes=False),
    )
    def build(src_h, dst_h, a_h, src_v, dst_v, acc_v, sem0, sem1):
        cid = lax.axis_index("c")
        sid = lax.axis_index("s")
        wid = cid * NS + sid
        sems = (sem0, sem1)
        NCH = E // ECH

        def start(ch, buf):
            pltpu.async_copy(src_h.at[pl.ds(ch * ECH, ECH)],
                             src_v.at[buf], sems[buf])
            pltpu.async_copy(dst_h.at[pl.ds(ch * ECH, ECH)],
                             dst_v.at[buf], sems[buf])

        def drain(buf):
            pltpu.make_async_copy(src_h.at[pl.ds(0, ECH)],
                                  src_v.at[buf], sems[buf]).wait()
            pltpu.make_async_copy(dst_h.at[pl.ds(0, ECH)],
                                  dst_v.at[buf], sems[buf]).wait()

        start(0, 0)
        for pi, (base_row, rpt) in enumerate(PASSES):
            lo = base_row + wid * rpt
            nw = rpt * N

            def zfill(i, _):
                acc_v[pl.ds(i * 16, 16)] = jnp.zeros((16,), F32)
                return 0
            lax.fori_loop(0, nw // 16, zfill, 0, unroll=8)

            ones = jnp.full((16,), 1.0, F32)
            tmask = jnp.full((16,), True)
            for q in range(rpt // 16):
                rel = q * 16 + lax.iota(jnp.int32, 16)
                plsc.addupdate_scatter(
                    acc_v, [rel * (N + 1) + lo], ones, mask=tmask)

            for ch in range(NCH):
                buf = ch % 2
                drain(buf)
                nxt = ch + 1
                if nxt == NCH and pi + 1 < len(PASSES):
                    start(0, nxt % 2)
                elif nxt < NCH:
                    start(nxt, nxt % 2)

                def grp(g, _, buf=buf):
                    sl = pl.ds(g * 16, 16)
                    s = src_v[buf, sl]
                    rel = dst_v[buf, sl] - lo
                    m = (rel >= 0) & (rel < rpt)
                    idx = jnp.where(m, rel * N + s, 0)
                    plsc.addupdate_scatter(
                        acc_v, [idx], jnp.full((16,), 1.0, F32), mask=m)
                    return 0
                lax.fori_loop(0, ECH // 16, grp, 0, unroll=8)

            pltpu.sync_copy(acc_v.at[pl.ds(0, nw)],
                            a_h.at[pl.ds(lo * N, nw)])

    return build(src1d, dst1d)


_TCPARAMS = pltpu.CompilerParams(dimension_semantics=("arbitrary",))
RB = 1024
G = N // RB


def _dinv(deg):
    return lax.rsqrt(deg + jnp.float32(1.0))


def _k1_body(base_ref, w1_ref, sens_ref, h_ref, s_ref):
    h_ref[...] = jnp.dot(base_ref[...], w1_ref[...],
                         preferred_element_type=F32)
    @pl.when(pl.program_id(0) == 0)
    def _():
        s_ref[...] = jnp.dot(sens_ref[...], w1_ref[...],
                             preferred_element_type=F32)


def _proj1(base_p, W1, sensor):
    return pl.pallas_call(
        _k1_body,
        grid=(G,),
        in_specs=[
            pl.BlockSpec((RB, EMBED), lambda i: (i, 0)),
            pl.BlockSpec((EMBED, HID), lambda i: (0, 0)),
            pl.BlockSpec((B, EMBED), lambda i: (0, 0)),
        ],
        out_specs=[
            pl.BlockSpec((RB, HID), lambda i: (i, 0)),
            pl.BlockSpec((B, HID), lambda i: (0, 0)),
        ],
        out_shape=[
            jax.ShapeDtypeStruct((N, HID), F32),
            jax.ShapeDtypeStruct((B, HID), F32),
        ],
        compiler_params=_TCPARAMS,
    )(base_p, W1, sensor)


def _mega_body(a_ref, h1_ref, sraw_ref,
               w2_ref, w3_ref, g1_ref, be1_ref, g2_ref, be2_ref,
               g3_ref, be3_ref, out_ref,
               h_ref, y_ref, s_ref, sum_ref, sq_ref, dv_ref, ab_ref):
    step = pl.program_id(0)
    phase = step // G
    i = step % G

    def dv_blk():
        return dv_ref[pl.ds(i * RB, RB)]

    def accum_stats(y):
        prev_s = jnp.where(i == 0, jnp.float32(0.0), sum_ref[...])
        prev_q = jnp.where(i == 0, jnp.float32(0.0), sq_ref[...])
        sum_ref[...] = prev_s + jnp.sum(y, axis=0, keepdims=True)
        sq_ref[...] = prev_q + jnp.sum(y * y, axis=0, keepdims=True)

    @pl.when(phase == 0)
    def _p0():
        a_blk = a_ref[pl.ds(i * RB, RB), :]
        dv = lax.rsqrt(jnp.sum(a_blk, axis=1))
        dv_ref[pl.ds(i * RB, RB)] = dv
        ab_ref[pl.ds(i * RB, RB), :] = a_blk.astype(jnp.bfloat16)
        h_ref[pl.ds(i * RB, RB), :HID] = (
            h1_ref[...] * dv[:, None]).astype(jnp.bfloat16)
        @pl.when(i == G - 1)
        def _():
            sel = lax.broadcasted_iota(jnp.int32, (1, RB), 1) == (
                N - 2 - (G - 1) * RB)
            d2046 = jnp.sum(jnp.where(sel, dv[None, :], jnp.float32(0.0)))
            s_ref[...] = sraw_ref[...] * d2046

    @pl.when(phase == 1)
    def _p1():
        dv = dv_blk()
        z = jnp.dot(ab_ref[pl.ds(i * RB, RB), :], h_ref[:, :HID],
                    preferred_element_type=F32) * dv[:, None]
        u = a_ref[pl.ds(i * RB, RB), N - 2:N - 1] * dv[:, None]
        s = s_ref[...]
        for b in range(B):
            y_ref[pl.ds(i * RB, RB), b * HID:(b + 1) * HID] = (
                z + u * s[b:b + 1, :])
        accum_stats(y_ref[pl.ds(i * RB, RB), :])

    def bn_elu_mm(w_ref_n, g_r, be_r):
        scale, shift = _bn_scale_shift2(sum_ref, sq_ref, g_r, be_r)
        dv = dv_blk()
        y = y_ref[pl.ds(i * RB, RB), :]
        w = w_ref_n[...]
        for b in range(B):
            yb = (y[:, b * HID:(b + 1) * HID] * scale[None, :]
                  + shift[None, :])
            xb = jnp.where(yb > 0, yb, jnp.exp(yb) - jnp.float32(1.0))
            h_ref[pl.ds(i * RB, RB), b * HID:(b + 1) * HID] = (
                jnp.dot(xb, w, preferred_element_type=F32)
                * dv[:, None]).astype(jnp.bfloat16)

    def agg():
        dv = dv_blk()
        y = jnp.dot(ab_ref[pl.ds(i * RB, RB), :], h_ref[...],
                    preferred_element_type=F32) * dv[:, None]
        y_ref[pl.ds(i * RB, RB), :] = y
        accum_stats(y)

    @pl.when(phase == 2)
    def _p2():
        bn_elu_mm(w2_ref, g1_ref, be1_ref)

    @pl.when(phase == 3)
    def _p3():
        agg()

    @pl.when(phase == 4)
    def _p4():
        bn_elu_mm(w3_ref, g2_ref, be2_ref)

    @pl.when(phase == 5)
    def _p5():
        agg()

    @pl.when(phase == 6)
    def _p6():
        scale, shift = _bn_scale_shift2(sum_ref, sq_ref, g3_ref, be3_ref)
        y = y_ref[pl.ds(i * RB, RB), :]
        for b in range(B):
            yb = (y[:, b * HID:(b + 1) * HID] * scale[None, :]
                  + shift[None, :])
            out_ref[:, b * HID:(b + 1) * HID] = jnp.where(
                yb > 0, yb, jnp.exp(yb) - jnp.float32(1.0))


def _bn_scale_shift2(sum_ref, sq_ref, g_ref, be_ref):
    cs = sum_ref[...].reshape(B, HID)
    cq = sq_ref[...].reshape(B, HID)
    inv_n = jnp.float32(1.0 / (B * N))
    mu = jnp.sum(cs, axis=0) * inv_n
    ex2 = jnp.sum(cq, axis=0) * inv_n
    var = ex2 - mu * mu
    scale = lax.rsqrt(var + BN_EPS) * g_ref[...]
    shift = be_ref[...] - mu * scale
    return scale, shift


def _mega(A, H1raw, Sraw, W2, W3, g1, be1, g2, be2, g3, be3):
    c0 = lambda s: (0, 0)
    c1 = lambda s: (0,)
    return pl.pallas_call(
        _mega_body,
        grid=(7 * G,),
        in_specs=[
            pl.BlockSpec((N, N), c0),
            pl.BlockSpec((RB, HID), lambda s: (jnp.minimum(s, G - 1), 0)),
            pl.BlockSpec((B, HID), c0),
            pl.BlockSpec((HID, HID), c0),
            pl.BlockSpec((HID, HID), c0),
            pl.BlockSpec((HID,), c1),
            pl.BlockSpec((HID,), c1),
            pl.BlockSpec((HID,), c1),
            pl.BlockSpec((HID,), c1),
            pl.BlockSpec((HID,), c1),
            pl.BlockSpec((HID,), c1),
        ],
        out_specs=pl.BlockSpec(
            (RB, B * HID), lambda s: (jnp.clip(s - 6 * G, 0, G - 1), 0)),
        out_shape=jax.ShapeDtypeStruct((N, B * HID), F32),
        scratch_shapes=[
            pltpu.VMEM((N, B * HID), jnp.bfloat16),
            pltpu.VMEM((N, B * HID), F32),
            pltpu.VMEM((B, HID), F32),
            pltpu.VMEM((1, B * HID), F32),
            pltpu.VMEM((1, B * HID), F32),
            pltpu.VMEM((N,), F32),
            pltpu.VMEM((N, N), jnp.bfloat16),
        ],
        compiler_params=_TCPARAMS,
    )(A, H1raw, Sraw, W2, W3, g1, be1, g2, be2, g3, be3)


def kernel(sensor_batch, base_vertices, edge_index,
           W1, b1, g1, be1, W2, b2, g2, be2, W3, b3, g3, be3):
    A = _sc_build_adjacency(edge_index[0], edge_index[1]).reshape(N, N)

    base_p = jnp.concatenate(
        [base_vertices, jnp.zeros((2, EMBED), F32)], axis=0)
    H1raw, Sraw = _proj1(base_p, W1, sensor_batch)
    X3 = _mega(A, H1raw, Sraw, W2, W3, g1, be1, g2, be2, g3, be3)

    return X3.reshape(N, B, HID).transpose(1, 0, 2)
